# baseline (speedup 1.0000x reference)
import jax
import jax.numpy as jnp
from jax import lax
from jax.experimental import pallas as pl
from jax.experimental.pallas import tpu as pltpu
from jax.experimental.pallas import tpu_sc as plsc

B = 16384
D = 64
NC = 2
NS = 16
NW = NC * NS
SEG = B // NW
RING = 6
HALF = SEG // 2


def _phase1(sv_u_hbm, perm_u_hbm, sv_i_hbm, perm_i_hbm,
            ut_hbm, it_hbm, uemb_hbm, iemb_hbm,
            sv_v, dcols_s, permv, outbuf, ring, sem, osem):
    wid = lax.axis_index("s") * NC + lax.axis_index("c")

    def run_pass(sv_hbm, perm_hbm, tab_hbm, emb_hbm):
        pltpu.sync_copy(sv_hbm.at[wid], sv_v)
        pltpu.sync_copy(perm_hbm.at[wid], permv)

        def dpass(g, carry):
            prev_c, cnt = carry
            v = sv_v[pl.ds(pl.multiple_of(g * 16, 16), 16)]
            for j in range(16):
                c = lax.shift_right_logical(v[j], 7)
                is_new = c != prev_c
                cnt = jnp.where(is_new, cnt + 1, cnt)

                @pl.when(is_new)
                def _():
                    dcols_s[cnt - 1] = c
                prev_c = c
            return prev_c, cnt

        lax.fori_loop(0, SEG // 16, dpass, (jnp.int32(-1), jnp.int32(0)))

        def fire(o):
            c = dcols_s[jnp.minimum(o, SEG - 1)]
            c = jnp.minimum(jnp.maximum(c, 0), 7812)
            off = pl.multiple_of(c * 128, 128)
            slot = lax.rem(o, RING)
            for s in range(RING):
                @pl.when(slot == s)
                def _():
                    pltpu.make_async_copy(
                        tab_hbm.at[:, pl.ds(off, 128)], ring.at[s], sem
                    ).start()

        def drain_one():
            pltpu.make_async_copy(
                tab_hbm.at[:, pl.ds(0, 128)], ring.at[0], sem).wait()

        for o in range(RING - 1):
            fire(jnp.int32(o))

        rows0 = lax.iota(jnp.int32, 16)

        def egroup(g, carry):
            prev_c, o = carry
            v = sv_v[pl.ds(pl.multiple_of(g * 16, 16), 16)]
            for j in range(16):
                c = lax.shift_right_logical(v[j], 7)
                l = lax.bitwise_and(v[j], 127)
                is_new = c != prev_c
                o = jnp.where(is_new, o + 1, o)

                @pl.when(is_new)
                def _():
                    drain_one()
                    fire(o + RING - 1)

                lvec = jnp.full((16,), l, jnp.int32)
                slot = lax.rem(o, RING)
                for s in range(RING):
                    @pl.when(slot == s)
                    def _():
                        for m in range(4):
                            gv = plsc.load_gather(
                                ring.at[s], [rows0 + m * 16, lvec])
                            outbuf[g * 16 + j, pl.ds(m * 16, 16)] = gv
                prev_c = c
            return prev_c, o

        last = lax.fori_loop(0, SEG // 16, egroup,
                             (jnp.int32(-1), jnp.int32(-1)))

        def cleanup(j, carry):
            drain_one()
            return carry
        lax.fori_loop(0, RING - 1, cleanup, last[1])

        for j in range(SEG // 128):
            pltpu.make_async_copy(
                outbuf.at[pl.ds(j * 128, 128)],
                emb_hbm.at[permv.at[j]], osem).start()
        for j in range(SEG // 128):
            pltpu.make_async_copy(
                outbuf.at[pl.ds(j * 128, 128)],
                emb_hbm.at[permv.at[j]], osem).wait()

    run_pass(sv_u_hbm, perm_u_hbm, ut_hbm, uemb_hbm)
    run_pass(sv_i_hbm, perm_i_hbm, it_hbm, iemb_hbm)


def _phase2(uemb_hbm, iemb_hbm, out_hbm, ubuf, ibuf, outv, sem):
    wid = lax.axis_index("s") * NC + lax.axis_index("c")
    base = wid * SEG
    lanes = lax.iota(jnp.int32, 16)

    for h in range(2):
        hb = h * HALF
        pltpu.make_async_copy(
            uemb_hbm.at[pl.ds(base + hb, HALF)], ubuf, sem).start()
        pltpu.make_async_copy(
            iemb_hbm.at[pl.ds(base + hb, HALF)], ibuf, sem).start()
        pltpu.make_async_copy(
            uemb_hbm.at[pl.ds(base + hb, HALF)], ubuf, sem).wait()
        pltpu.make_async_copy(
            iemb_hbm.at[pl.ds(base + hb, HALF)], ibuf, sem).wait()

        def group(g, carry):
            gb = pl.multiple_of(g * 16, 16)
            vec = jnp.zeros((16,), jnp.float32)
            for l in range(16):
                r = gb + l
                p0 = ubuf[r, pl.ds(0, 16)] * ibuf[r, pl.ds(0, 16)]
                p1 = ubuf[r, pl.ds(16, 16)] * ibuf[r, pl.ds(16, 16)]
                p2 = ubuf[r, pl.ds(32, 16)] * ibuf[r, pl.ds(32, 16)]
                p3 = ubuf[r, pl.ds(48, 16)] * ibuf[r, pl.ds(48, 16)]
                acc = (p0 + p1) + (p2 + p3)
                s = jnp.sum(acc)
                vec = jnp.where(lanes == l, s, vec)
            outv[pl.ds(pl.multiple_of(hb + gb, 16), 16)] = vec
            return carry

        lax.fori_loop(0, HALF // 16, group, 0)

    pltpu.sync_copy(outv, out_hbm.at[wid])


def _prep(v):
    pos = jnp.arange(B, dtype=jnp.int32)
    sv, p = lax.sort((v, pos), num_keys=1)
    return sv.reshape(NW, SEG), p.reshape(NW, SEG // 128, 128)


@jax.jit
def kernel(x, uid_table, iid_table):
    vu = x[:, 0].astype(jnp.int32)
    vi = x[:, 1].astype(jnp.int32)
    sv_u, perm_u = _prep(vu)
    sv_i, perm_i = _prep(vi)
    ut_t = uid_table.T
    it_t = iid_table.T

    mesh = plsc.VectorSubcoreMesh(
        core_axis_name="c", subcore_axis_name="s",
        num_cores=NC, num_subcores=NS)
    cp = pltpu.CompilerParams(
        needs_layout_passes=False, use_tc_tiling_on_sc=True)

    uemb, iemb = pl.kernel(
        _phase1,
        out_type=(jax.ShapeDtypeStruct((B, 128), jnp.float32),
                  jax.ShapeDtypeStruct((B, 128), jnp.float32)),
        mesh=mesh,
        compiler_params=cp,
        scratch_types=[
            pltpu.VMEM((SEG,), jnp.int32),
            pltpu.SMEM((SEG,), jnp.int32),
            pltpu.VMEM((SEG // 128, 128), jnp.int32),
            pltpu.VMEM((SEG, 128), jnp.float32),
            pltpu.VMEM((RING, D, 128), jnp.float32),
            pltpu.SemaphoreType.DMA,
            pltpu.SemaphoreType.DMA,
        ],
    )(sv_u, perm_u, sv_i, perm_i, ut_t, it_t)

    out = pl.kernel(
        _phase2,
        out_type=jax.ShapeDtypeStruct((NW, SEG), jnp.float32),
        mesh=mesh,
        compiler_params=cp,
        scratch_types=[
            pltpu.VMEM((HALF, 128), jnp.float32),
            pltpu.VMEM((HALF, 128), jnp.float32),
            pltpu.VMEM((SEG,), jnp.float32),
            pltpu.SemaphoreType.DMA,
        ],
    )(uemb, iemb)
    return out.reshape(B)


# speedup vs baseline: 1.1282x; 1.1282x over previous
import jax
import jax.numpy as jnp
from jax import lax
from jax.experimental import pallas as pl
from jax.experimental.pallas import tpu as pltpu
from jax.experimental.pallas import tpu_sc as plsc

B = 16384
D = 64
NC = 2
NS = 16
NW = NC * NS
SEG = B // NW
RING = 4
HALF = SEG // 2


def _phase1(sv_u_hbm, perm_u_hbm, sv_i_hbm, perm_i_hbm,
            ut_hbm, it_hbm, uemb_hbm, iemb_hbm,
            sv_v, dcols_s, permv, outbuf, ring, sem, osem):
    wid = lax.axis_index("s") * NC + lax.axis_index("c")

    def run_pass(sv_hbm, perm_hbm, tab_hbm, emb_hbm):
        pltpu.sync_copy(sv_hbm.at[wid], sv_v)
        pltpu.sync_copy(perm_hbm.at[wid], permv)

        def dpass(g, carry):
            prev_c, cnt = carry
            v = sv_v[pl.ds(pl.multiple_of(g * 16, 16), 16)]
            for j in range(16):
                c = lax.shift_right_logical(v[j], 7)
                is_new = c != prev_c
                cnt = jnp.where(is_new, cnt + 1, cnt)

                @pl.when(is_new)
                def _():
                    dcols_s[cnt - 1] = c
                prev_c = c
            return prev_c, cnt

        lax.fori_loop(0, SEG // 16, dpass, (jnp.int32(-1), jnp.int32(0)))

        def fire(o):
            c = dcols_s[jnp.minimum(o, SEG - 1)]
            c = jnp.minimum(jnp.maximum(c, 0), 7812)
            off = pl.multiple_of(c * 128, 128)
            slot = lax.rem(o, RING)
            pltpu.make_async_copy(
                tab_hbm.at[:, pl.ds(off, 128)], ring.at[slot], sem).start()

        def drain_one():
            pltpu.make_async_copy(
                tab_hbm.at[:, pl.ds(0, 128)], ring.at[0], sem).wait()

        for o in range(RING - 1):
            fire(jnp.int32(o))

        rows0 = lax.iota(jnp.int32, 16)

        def egroup(g, carry):
            prev_c, o = carry
            v = sv_v[pl.ds(pl.multiple_of(g * 16, 16), 16)]
            for j in range(16):
                c = lax.shift_right_logical(v[j], 7)
                l = lax.bitwise_and(v[j], 127)
                is_new = c != prev_c
                o = jnp.where(is_new, o + 1, o)

                @pl.when(is_new)
                def _():
                    drain_one()
                    fire(o + RING - 1)

                lvec = jnp.full((16,), l, jnp.int32)
                svec = jnp.full((16,), lax.rem(o, RING), jnp.int32)
                for m in range(4):
                    gv = plsc.load_gather(
                        ring, [svec, rows0 + m * 16, lvec])
                    outbuf[g * 16 + j, pl.ds(m * 16, 16)] = gv
                prev_c = c
            return prev_c, o

        last = lax.fori_loop(0, SEG // 16, egroup,
                             (jnp.int32(-1), jnp.int32(-1)))

        def cleanup(j, carry):
            drain_one()
            return carry
        lax.fori_loop(0, RING - 1, cleanup, last[1])

        for j in range(SEG // 128):
            pltpu.make_async_copy(
                outbuf.at[pl.ds(j * 128, 128)],
                emb_hbm.at[permv.at[j]], osem).start()
        for j in range(SEG // 128):
            pltpu.make_async_copy(
                outbuf.at[pl.ds(j * 128, 128)],
                emb_hbm.at[permv.at[j]], osem).wait()

    run_pass(sv_u_hbm, perm_u_hbm, ut_hbm, uemb_hbm)
    run_pass(sv_i_hbm, perm_i_hbm, it_hbm, iemb_hbm)


def _phase2(uemb_hbm, iemb_hbm, out_hbm, ubuf, ibuf, outv, sem):
    wid = lax.axis_index("s") * NC + lax.axis_index("c")
    base = wid * SEG
    lanes = lax.iota(jnp.int32, 16)

    for h in range(2):
        hb = h * HALF
        pltpu.make_async_copy(
            uemb_hbm.at[pl.ds(base + hb, HALF)], ubuf, sem).start()
        pltpu.make_async_copy(
            iemb_hbm.at[pl.ds(base + hb, HALF)], ibuf, sem).start()
        pltpu.make_async_copy(
            uemb_hbm.at[pl.ds(base + hb, HALF)], ubuf, sem).wait()
        pltpu.make_async_copy(
            iemb_hbm.at[pl.ds(base + hb, HALF)], ibuf, sem).wait()

        def group(g, carry):
            gb = pl.multiple_of(g * 16, 16)
            vec = jnp.zeros((16,), jnp.float32)
            for l in range(16):
                r = gb + l
                p0 = ubuf[r, pl.ds(0, 16)] * ibuf[r, pl.ds(0, 16)]
                p1 = ubuf[r, pl.ds(16, 16)] * ibuf[r, pl.ds(16, 16)]
                p2 = ubuf[r, pl.ds(32, 16)] * ibuf[r, pl.ds(32, 16)]
                p3 = ubuf[r, pl.ds(48, 16)] * ibuf[r, pl.ds(48, 16)]
                acc = (p0 + p1) + (p2 + p3)
                s = jnp.sum(acc)
                vec = jnp.where(lanes == l, s, vec)
            outv[pl.ds(pl.multiple_of(hb + gb, 16), 16)] = vec
            return carry

        lax.fori_loop(0, HALF // 16, group, 0)

    pltpu.sync_copy(outv, out_hbm.at[wid])


def _prep(v):
    pos = jnp.arange(B, dtype=jnp.int32)
    sv, p = lax.sort((v, pos), num_keys=1)
    return sv.reshape(NW, SEG), p.reshape(NW, SEG // 128, 128)


@jax.jit
def kernel(x, uid_table, iid_table):
    vu = x[:, 0].astype(jnp.int32)
    vi = x[:, 1].astype(jnp.int32)
    sv_u, perm_u = _prep(vu)
    sv_i, perm_i = _prep(vi)
    ut_t = uid_table.T
    it_t = iid_table.T

    mesh = plsc.VectorSubcoreMesh(
        core_axis_name="c", subcore_axis_name="s",
        num_cores=NC, num_subcores=NS)
    cp = pltpu.CompilerParams(
        needs_layout_passes=False, use_tc_tiling_on_sc=True)

    uemb, iemb = pl.kernel(
        _phase1,
        out_type=(jax.ShapeDtypeStruct((B, 128), jnp.float32),
                  jax.ShapeDtypeStruct((B, 128), jnp.float32)),
        mesh=mesh,
        compiler_params=cp,
        scratch_types=[
            pltpu.VMEM((SEG,), jnp.int32),
            pltpu.SMEM((SEG,), jnp.int32),
            pltpu.VMEM((SEG // 128, 128), jnp.int32),
            pltpu.VMEM((SEG, 128), jnp.float32),
            pltpu.VMEM((RING, D, 128), jnp.float32),
            pltpu.SemaphoreType.DMA,
            pltpu.SemaphoreType.DMA,
        ],
    )(sv_u, perm_u, sv_i, perm_i, ut_t, it_t)

    out = pl.kernel(
        _phase2,
        out_type=jax.ShapeDtypeStruct((NW, SEG), jnp.float32),
        mesh=mesh,
        compiler_params=cp,
        scratch_types=[
            pltpu.VMEM((HALF, 128), jnp.float32),
            pltpu.VMEM((HALF, 128), jnp.float32),
            pltpu.VMEM((SEG,), jnp.float32),
            pltpu.SemaphoreType.DMA,
        ],
    )(uemb, iemb)
    return out.reshape(B)


# speedup vs baseline: 1.2104x; 1.0729x over previous
import jax
import jax.numpy as jnp
from jax import lax
from jax.experimental import pallas as pl
from jax.experimental.pallas import tpu as pltpu
from jax.experimental.pallas import tpu_sc as plsc

B = 16384
D = 64
NC = 2
NS = 16
NW = NC * NS
SEG = B // NW
RING = 6
HALF = SEG // 2


def _phase1(sv_u_hbm, perm_u_hbm, sv_i_hbm, perm_i_hbm,
            ut_hbm, it_hbm, uemb_hbm, iemb_hbm,
            sv_v, dcols_s, permv, outbuf, ring, sem, osem):
    wid = lax.axis_index("s") * NC + lax.axis_index("c")

    def run_pass(sv_hbm, perm_hbm, tab_hbm, emb_hbm):
        pltpu.sync_copy(sv_hbm.at[wid], sv_v)
        pltpu.sync_copy(perm_hbm.at[wid], permv)

        def dpass(g, carry):
            prev_c, cnt = carry
            v = sv_v[pl.ds(pl.multiple_of(g * 16, 16), 16)]
            for j in range(16):
                c = lax.shift_right_logical(v[j], 7)
                is_new = c != prev_c
                cnt = jnp.where(is_new, cnt + 1, cnt)

                @pl.when(is_new)
                def _():
                    dcols_s[cnt - 1] = c
                prev_c = c
            return prev_c, cnt

        lax.fori_loop(0, SEG // 16, dpass, (jnp.int32(-1), jnp.int32(0)))

        def fire(o):
            c = dcols_s[jnp.minimum(o, SEG - 1)]
            c = jnp.minimum(jnp.maximum(c, 0), 7812)
            off = pl.multiple_of(c * 128, 128)
            slot = lax.rem(o, RING)
            pltpu.make_async_copy(
                tab_hbm.at[:, pl.ds(off, 128)], ring.at[slot], sem).start()

        def drain_one():
            pltpu.make_async_copy(
                tab_hbm.at[:, pl.ds(0, 128)], ring.at[0], sem).wait()

        for o in range(RING - 1):
            fire(jnp.int32(o))

        rows0 = lax.iota(jnp.int32, 16)

        def egroup(g, carry):
            prev_c, o = carry
            v = sv_v[pl.ds(pl.multiple_of(g * 16, 16), 16)]
            for j in range(16):
                c = lax.shift_right_logical(v[j], 7)
                l = lax.bitwise_and(v[j], 127)
                is_new = c != prev_c
                o = jnp.where(is_new, o + 1, o)

                @pl.when(is_new)
                def _():
                    drain_one()
                    fire(o + RING - 1)

                lvec = jnp.full((16,), l, jnp.int32)
                svec = jnp.full((16,), lax.rem(o, RING), jnp.int32)
                for m in range(4):
                    gv = plsc.load_gather(
                        ring, [svec, rows0 + m * 16, lvec])
                    outbuf[g * 16 + j, pl.ds(m * 16, 16)] = gv
                prev_c = c
            return prev_c, o

        last = lax.fori_loop(0, SEG // 16, egroup,
                             (jnp.int32(-1), jnp.int32(-1)))

        def cleanup(j, carry):
            drain_one()
            return carry
        lax.fori_loop(0, RING - 1, cleanup, last[1])

        for j in range(SEG // 128):
            pltpu.make_async_copy(
                outbuf.at[pl.ds(j * 128, 128)],
                emb_hbm.at[permv.at[j]], osem).start()
        for j in range(SEG // 128):
            pltpu.make_async_copy(
                outbuf.at[pl.ds(j * 128, 128)],
                emb_hbm.at[permv.at[j]], osem).wait()

    run_pass(sv_u_hbm, perm_u_hbm, ut_hbm, uemb_hbm)
    run_pass(sv_i_hbm, perm_i_hbm, it_hbm, iemb_hbm)


def _phase2(uemb_hbm, iemb_hbm, out_hbm, ubuf, ibuf, outv, sem):
    wid = lax.axis_index("s") * NC + lax.axis_index("c")
    base = wid * SEG
    lanes = lax.iota(jnp.int32, 16)

    for h in range(2):
        hb = h * HALF
        pltpu.make_async_copy(
            uemb_hbm.at[pl.ds(base + hb, HALF)], ubuf, sem).start()
        pltpu.make_async_copy(
            iemb_hbm.at[pl.ds(base + hb, HALF)], ibuf, sem).start()
        pltpu.make_async_copy(
            uemb_hbm.at[pl.ds(base + hb, HALF)], ubuf, sem).wait()
        pltpu.make_async_copy(
            iemb_hbm.at[pl.ds(base + hb, HALF)], ibuf, sem).wait()

        def group(g, carry):
            gb = pl.multiple_of(g * 16, 16)
            vec = jnp.zeros((16,), jnp.float32)
            for l in range(16):
                r = gb + l
                p0 = ubuf[r, pl.ds(0, 16)] * ibuf[r, pl.ds(0, 16)]
                p1 = ubuf[r, pl.ds(16, 16)] * ibuf[r, pl.ds(16, 16)]
                p2 = ubuf[r, pl.ds(32, 16)] * ibuf[r, pl.ds(32, 16)]
                p3 = ubuf[r, pl.ds(48, 16)] * ibuf[r, pl.ds(48, 16)]
                acc = (p0 + p1) + (p2 + p3)
                s = jnp.sum(acc)
                vec = jnp.where(lanes == l, s, vec)
            outv[pl.ds(pl.multiple_of(hb + gb, 16), 16)] = vec
            return carry

        lax.fori_loop(0, HALF // 16, group, 0)

    pltpu.sync_copy(outv, out_hbm.at[wid])


def _prep(v):
    pos = jnp.arange(B, dtype=jnp.int32)
    sv, p = lax.sort((v, pos), num_keys=1)
    return sv.reshape(NW, SEG), p.reshape(NW, SEG // 128, 128)


@jax.jit
def kernel(x, uid_table, iid_table):
    vu = x[:, 0].astype(jnp.int32)
    vi = x[:, 1].astype(jnp.int32)
    sv_u, perm_u = _prep(vu)
    sv_i, perm_i = _prep(vi)
    ut_t = uid_table.T
    it_t = iid_table.T

    mesh = plsc.VectorSubcoreMesh(
        core_axis_name="c", subcore_axis_name="s",
        num_cores=NC, num_subcores=NS)
    cp = pltpu.CompilerParams(
        needs_layout_passes=False, use_tc_tiling_on_sc=True)

    uemb, iemb = pl.kernel(
        _phase1,
        out_type=(jax.ShapeDtypeStruct((B, 128), jnp.float32),
                  jax.ShapeDtypeStruct((B, 128), jnp.float32)),
        mesh=mesh,
        compiler_params=cp,
        scratch_types=[
            pltpu.VMEM((SEG,), jnp.int32),
            pltpu.SMEM((SEG,), jnp.int32),
            pltpu.VMEM((SEG // 128, 128), jnp.int32),
            pltpu.VMEM((SEG, 128), jnp.float32),
            pltpu.VMEM((RING, D, 128), jnp.float32),
            pltpu.SemaphoreType.DMA,
            pltpu.SemaphoreType.DMA,
        ],
    )(sv_u, perm_u, sv_i, perm_i, ut_t, it_t)

    out = pl.kernel(
        _phase2,
        out_type=jax.ShapeDtypeStruct((NW, SEG), jnp.float32),
        mesh=mesh,
        compiler_params=cp,
        scratch_types=[
            pltpu.VMEM((HALF, 128), jnp.float32),
            pltpu.VMEM((HALF, 128), jnp.float32),
            pltpu.VMEM((SEG,), jnp.float32),
            pltpu.SemaphoreType.DMA,
        ],
    )(uemb, iemb)
    return out.reshape(B)
